# baseline (speedup 1.0000x reference)
import jax
import jax.numpy as jnp
from jax.experimental import pallas as pl

_PE_L = 6
_H = 32
_B = 4096
_G = 4

_S1 = 3.141592653589793
_S3 = -5.167712780049970
_S5 = 2.550164039877345
_S7 = -0.5992645293207921
_S9 = 0.0821458866111282
_S11 = -0.0073704309457144
_C0 = 1.0
_C2 = -4.934802200544679
_C4 = 4.058712126416768
_C6 = -1.3352627688545895
_C8 = 0.2353306303588932
_C10 = -0.0258068327360992


def _sincos_pi(t):
    n = jnp.floor(t + 0.5)
    r = t - n
    z = r * r
    ps = ((((_S11 * z + _S9) * z + _S7) * z + _S5) * z + _S3) * z + _S1
    ps = ps * r
    pc = ((((_C10 * z + _C8) * z + _C6) * z + _C4) * z + _C2) * z + _C0
    an = jnp.abs(n)
    sgn = 1.0 - 2.0 * (an - 2.0 * jnp.floor(an * 0.5))
    return ps * sgn, pc * sgn


def _volume_body(xyzT_ref, ynmT_ref, scale_ref, off_ref, w1_ref, b1_ref,
                 w2_ref, b2_ref, dT_ref, cT_ref):
    B = _B
    C = B // _G
    t = xyzT_ref[:] * scale_ref[:] + off_ref[:]
    inb = (t >= -1.0) & (t <= 1.0)
    mask = inb[0:1] & inb[1:2] & inb[2:3]

    s, c = _sincos_pi(t)
    feats = [t, s, c]
    for _ in range(1, _PE_L):
        s, c = 2.0 * s * c, 1.0 - 2.0 * s * s
        feats.append(s)
        feats.append(c)
    pe = jnp.concatenate(feats, axis=0)

    pe_g = jnp.concatenate([pe[:, i * C:(i + 1) * C] for i in range(_G)],
                           axis=0).astype(jnp.bfloat16)
    f_g = jnp.dot(w1_ref[:], pe_g,
                  preferred_element_type=jnp.float32)
    f_g = jnp.maximum(f_g + b1_ref[:], 0.0)

    ynmT = ynmT_ref[:].astype(jnp.bfloat16)
    z_parts = []
    for i in range(_G):
        z_parts.append(f_g[_H * i:_H * (i + 1), :].astype(jnp.bfloat16))
        z_parts.append(ynmT[:, i * C:(i + 1) * C])
    z_g = jnp.concatenate(z_parts, axis=0)
    o_g = jnp.dot(w2_ref[:], z_g,
                  preferred_element_type=jnp.float32) + b2_ref[:]
    o = jnp.concatenate([o_g[4 * i:4 * (i + 1), :] for i in range(_G)],
                        axis=1)

    row = jax.lax.broadcasted_iota(jnp.int32, (4, B), 0)
    sig = 1.0 / (1.0 + jnp.exp(-o))
    sp = jnp.maximum(o, 0.0) + jnp.log1p(jnp.exp(-jnp.abs(o)))
    act = jnp.where(row == 0, sp, sig)
    act = jnp.where(mask, act, 0.0)
    dT_ref[:] = act[0:1, :]
    cT_ref[:] = act[1:4, :]


def kernel(xyz, ynm, aabb, W1, b1, Wd, bd, Wc, bc):
    N, S, _ = xyz.shape
    P = N * S
    B, G = _B, _G

    xyzT = jax.lax.optimization_barrier(xyz.reshape(P, 3)).T
    ynmT = jax.lax.optimization_barrier(ynm.reshape(P, 16)).T
    rng = aabb[1] - aabb[0]
    scale = (2.0 / rng).reshape(3, 1)
    off = (-2.0 * aabb[0] / rng - 1.0).reshape(3, 1)

    W1T = W1.T
    w1bd = jax.scipy.linalg.block_diag(*([W1T] * G)).astype(jnp.bfloat16)
    b1t = jnp.tile(b1, G).reshape(G * _H, 1)
    W2 = jnp.concatenate(
        [jnp.concatenate([Wd, jnp.zeros((16, 1), jnp.float32)], axis=0), Wc],
        axis=1)
    w2bd = jax.scipy.linalg.block_diag(*([W2.T] * G)).astype(jnp.bfloat16)
    b2t = jnp.tile(jnp.concatenate([bd, bc]), G).reshape(4 * G, 1)

    grid = P // B
    dT, cT = pl.pallas_call(
        _volume_body,
        grid=(grid,),
        in_specs=[
            pl.BlockSpec((3, B), lambda i: (0, i)),
            pl.BlockSpec((16, B), lambda i: (0, i)),
            pl.BlockSpec((3, 1), lambda i: (0, 0)),
            pl.BlockSpec((3, 1), lambda i: (0, 0)),
            pl.BlockSpec(w1bd.shape, lambda i: (0, 0)),
            pl.BlockSpec((G * _H, 1), lambda i: (0, 0)),
            pl.BlockSpec(w2bd.shape, lambda i: (0, 0)),
            pl.BlockSpec((4 * G, 1), lambda i: (0, 0)),
        ],
        out_specs=[
            pl.BlockSpec((1, B), lambda i: (0, i)),
            pl.BlockSpec((3, B), lambda i: (0, i)),
        ],
        out_shape=[
            jax.ShapeDtypeStruct((1, P), jnp.float32),
            jax.ShapeDtypeStruct((3, P), jnp.float32),
        ],
    )(xyzT, ynmT, scale, off, w1bd, b1t, w2bd, b2t)

    density = dT.reshape(N, S, 1)
    color = jax.lax.optimization_barrier(cT).T.reshape(N, S, 3)
    return density, color


# speedup vs baseline: 1.1548x; 1.1548x over previous
import jax
import jax.numpy as jnp
from jax.experimental import pallas as pl

_PE_L = 6
_H = 32
_B = 8192
_G = 4

_S1 = 3.141592653589793
_S3 = -5.167712780049970
_S5 = 2.550164039877345
_S7 = -0.5992645293207921
_S9 = 0.0821458866111282
_S11 = -0.0073704309457144
_C0 = 1.0
_C2 = -4.934802200544679
_C4 = 4.058712126416768
_C6 = -1.3352627688545895
_C8 = 0.2353306303588932
_C10 = -0.0258068327360992


def _sincos_pi(t):
    n = jnp.floor(t + 0.5)
    r = t - n
    z = r * r
    ps = ((((_S11 * z + _S9) * z + _S7) * z + _S5) * z + _S3) * z + _S1
    ps = ps * r
    pc = ((((_C10 * z + _C8) * z + _C6) * z + _C4) * z + _C2) * z + _C0
    an = jnp.abs(n)
    sgn = 1.0 - 2.0 * (an - 2.0 * jnp.floor(an * 0.5))
    return ps * sgn, pc * sgn


def _volume_body(xyzT_ref, ynmT_ref, scale_ref, off_ref, w1_ref, b1_ref,
                 w2_ref, b2_ref, dT_ref, cT_ref):
    B = _B
    C = B // _G
    t = xyzT_ref[:] * scale_ref[:] + off_ref[:]
    inb = (t >= -1.0) & (t <= 1.0)
    mask = inb[0:1] & inb[1:2] & inb[2:3]

    s, c = _sincos_pi(t)
    feats = [t, s, c]
    for _ in range(1, _PE_L):
        s, c = 2.0 * s * c, 1.0 - 2.0 * s * s
        feats.append(s)
        feats.append(c)
    pe = jnp.concatenate(feats, axis=0)

    pe_g = jnp.concatenate([pe[:, i * C:(i + 1) * C] for i in range(_G)],
                           axis=0).astype(jnp.bfloat16)
    f_g = jnp.dot(w1_ref[:], pe_g,
                  preferred_element_type=jnp.float32)
    f_g = jnp.maximum(f_g + b1_ref[:], 0.0)

    ynmT = ynmT_ref[:].astype(jnp.bfloat16)
    z_parts = []
    for i in range(_G):
        z_parts.append(f_g[_H * i:_H * (i + 1), :].astype(jnp.bfloat16))
        z_parts.append(ynmT[:, i * C:(i + 1) * C])
    z_g = jnp.concatenate(z_parts, axis=0)
    o_g = jnp.dot(w2_ref[:], z_g,
                  preferred_element_type=jnp.float32) + b2_ref[:]
    o = jnp.concatenate([o_g[4 * i:4 * (i + 1), :] for i in range(_G)],
                        axis=1)

    od = o[0:1, :]
    sp = jnp.maximum(od, 0.0) + jnp.log1p(jnp.exp(-jnp.abs(od)))
    oc = o[1:4, :]
    sig = 1.0 / (1.0 + jnp.exp(-oc))
    dT_ref[:] = jnp.where(mask, sp, 0.0)
    cT_ref[:] = jnp.where(mask, sig, 0.0)


def kernel(xyz, ynm, aabb, W1, b1, Wd, bd, Wc, bc):
    N, S, _ = xyz.shape
    P = N * S
    B, G = _B, _G

    xyzT = xyz.transpose(2, 0, 1).reshape(3, P)
    ynmT = ynm.transpose(2, 0, 1).reshape(16, P)
    rng = aabb[1] - aabb[0]
    scale = (2.0 / rng).reshape(3, 1)
    off = (-2.0 * aabb[0] / rng - 1.0).reshape(3, 1)

    W1T = W1.T
    w1bd = jax.scipy.linalg.block_diag(*([W1T] * G)).astype(jnp.bfloat16)
    b1t = jnp.tile(b1, G).reshape(G * _H, 1)
    W2 = jnp.concatenate(
        [jnp.concatenate([Wd, jnp.zeros((16, 1), jnp.float32)], axis=0), Wc],
        axis=1)
    w2bd = jax.scipy.linalg.block_diag(*([W2.T] * G)).astype(jnp.bfloat16)
    b2t = jnp.tile(jnp.concatenate([bd, bc]), G).reshape(4 * G, 1)

    grid = P // B
    dT, cT = pl.pallas_call(
        _volume_body,
        grid=(grid,),
        in_specs=[
            pl.BlockSpec((3, B), lambda i: (0, i)),
            pl.BlockSpec((16, B), lambda i: (0, i)),
            pl.BlockSpec((3, 1), lambda i: (0, 0)),
            pl.BlockSpec((3, 1), lambda i: (0, 0)),
            pl.BlockSpec(w1bd.shape, lambda i: (0, 0)),
            pl.BlockSpec((G * _H, 1), lambda i: (0, 0)),
            pl.BlockSpec(w2bd.shape, lambda i: (0, 0)),
            pl.BlockSpec((4 * G, 1), lambda i: (0, 0)),
        ],
        out_specs=[
            pl.BlockSpec((1, B), lambda i: (0, i)),
            pl.BlockSpec((3, B), lambda i: (0, i)),
        ],
        out_shape=[
            jax.ShapeDtypeStruct((1, P), jnp.float32),
            jax.ShapeDtypeStruct((3, P), jnp.float32),
        ],
    )(xyzT, ynmT, scale, off, w1bd, b1t, w2bd, b2t)

    density = dT.reshape(N, S, 1)
    color = cT.reshape(3, N, S).transpose(1, 2, 0)
    return density, color


# speedup vs baseline: 1.1861x; 1.0271x over previous
import jax
import jax.numpy as jnp
from jax.experimental import pallas as pl

_PE_L = 6
_H = 32
_B = 16384
_G = 4

_S1 = 3.141592653589793
_S3 = -5.167712780049970
_S5 = 2.550164039877345
_S7 = -0.5992645293207921
_S9 = 0.0821458866111282
_S11 = -0.0073704309457144
_C0 = 1.0
_C2 = -4.934802200544679
_C4 = 4.058712126416768
_C6 = -1.3352627688545895
_C8 = 0.2353306303588932
_C10 = -0.0258068327360992


def _sincos_pi(t):
    n = jnp.floor(t + 0.5)
    r = t - n
    z = r * r
    ps = ((((_S11 * z + _S9) * z + _S7) * z + _S5) * z + _S3) * z + _S1
    ps = ps * r
    pc = ((((_C10 * z + _C8) * z + _C6) * z + _C4) * z + _C2) * z + _C0
    an = jnp.abs(n)
    sgn = 1.0 - 2.0 * (an - 2.0 * jnp.floor(an * 0.5))
    return ps * sgn, pc * sgn


def _volume_body(xyzT_ref, ynmT_ref, scale_ref, off_ref, w1_ref, b1_ref,
                 w2_ref, b2_ref, dT_ref, cT_ref):
    B = _B
    C = B // _G
    t = xyzT_ref[:] * scale_ref[:] + off_ref[:]
    inb = (t >= -1.0) & (t <= 1.0)
    mask = inb[0:1] & inb[1:2] & inb[2:3]

    s, c = _sincos_pi(t)
    feats = [t, s, c]
    for _ in range(1, _PE_L):
        s, c = 2.0 * s * c, 1.0 - 2.0 * s * s
        feats.append(s)
        feats.append(c)
    pe = jnp.concatenate(feats, axis=0)

    pe_g = jnp.concatenate([pe[:, i * C:(i + 1) * C] for i in range(_G)],
                           axis=0).astype(jnp.bfloat16)
    f_g = jnp.dot(w1_ref[:], pe_g,
                  preferred_element_type=jnp.float32)
    f_g = jnp.maximum(f_g + b1_ref[:], 0.0)

    ynmT = ynmT_ref[:].astype(jnp.bfloat16)
    z_parts = []
    for i in range(_G):
        z_parts.append(f_g[_H * i:_H * (i + 1), :].astype(jnp.bfloat16))
        z_parts.append(ynmT[:, i * C:(i + 1) * C])
    z_g = jnp.concatenate(z_parts, axis=0)
    o_g = jnp.dot(w2_ref[:], z_g,
                  preferred_element_type=jnp.float32) + b2_ref[:]
    o = jnp.concatenate([o_g[4 * i:4 * (i + 1), :] for i in range(_G)],
                        axis=1)

    od = o[0:1, :]
    sp = jnp.maximum(od, 0.0) + jnp.log1p(jnp.exp(-jnp.abs(od)))
    oc = o[1:4, :]
    sig = 1.0 / (1.0 + jnp.exp(-oc))
    dT_ref[:] = jnp.where(mask, sp, 0.0)
    cT_ref[:] = jnp.where(mask, sig, 0.0)


def kernel(xyz, ynm, aabb, W1, b1, Wd, bd, Wc, bc):
    N, S, _ = xyz.shape
    P = N * S
    B, G = _B, _G

    xyzT = xyz.transpose(2, 0, 1).reshape(3, P)
    ynmT = ynm.transpose(2, 0, 1).reshape(16, P)
    rng = aabb[1] - aabb[0]
    scale = (2.0 / rng).reshape(3, 1)
    off = (-2.0 * aabb[0] / rng - 1.0).reshape(3, 1)

    W1T = W1.T
    w1bd = jax.scipy.linalg.block_diag(*([W1T] * G)).astype(jnp.bfloat16)
    b1t = jnp.tile(b1, G).reshape(G * _H, 1)
    W2 = jnp.concatenate(
        [jnp.concatenate([Wd, jnp.zeros((16, 1), jnp.float32)], axis=0), Wc],
        axis=1)
    w2bd = jax.scipy.linalg.block_diag(*([W2.T] * G)).astype(jnp.bfloat16)
    b2t = jnp.tile(jnp.concatenate([bd, bc]), G).reshape(4 * G, 1)

    grid = P // B
    dT, cT = pl.pallas_call(
        _volume_body,
        grid=(grid,),
        in_specs=[
            pl.BlockSpec((3, B), lambda i: (0, i)),
            pl.BlockSpec((16, B), lambda i: (0, i)),
            pl.BlockSpec((3, 1), lambda i: (0, 0)),
            pl.BlockSpec((3, 1), lambda i: (0, 0)),
            pl.BlockSpec(w1bd.shape, lambda i: (0, 0)),
            pl.BlockSpec((G * _H, 1), lambda i: (0, 0)),
            pl.BlockSpec(w2bd.shape, lambda i: (0, 0)),
            pl.BlockSpec((4 * G, 1), lambda i: (0, 0)),
        ],
        out_specs=[
            pl.BlockSpec((1, B), lambda i: (0, i)),
            pl.BlockSpec((3, B), lambda i: (0, i)),
        ],
        out_shape=[
            jax.ShapeDtypeStruct((1, P), jnp.float32),
            jax.ShapeDtypeStruct((3, P), jnp.float32),
        ],
    )(xyzT, ynmT, scale, off, w1bd, b1t, w2bd, b2t)

    density = dT.reshape(N, S, 1)
    color = cT.reshape(3, N, S).transpose(1, 2, 0)
    return density, color


# speedup vs baseline: 1.2577x; 1.0604x over previous
import jax
import jax.numpy as jnp
from jax.experimental import pallas as pl

_PE_L = 6
_H = 32
_B = 16384
_G = 4

_S1 = 3.141592653589793
_S3 = -5.167712780049970
_S5 = 2.550164039877345
_S7 = -0.5992645293207921
_S9 = 0.0821458866111282
_S11 = -0.0073704309457144
_C0 = 1.0
_C2 = -4.934802200544679
_C4 = 4.058712126416768
_C6 = -1.3352627688545895
_C8 = 0.2353306303588932
_C10 = -0.0258068327360992


def _sincos_pi(t):
    n = jnp.floor(t + 0.5)
    r = t - n
    z = r * r
    ps = ((((_S11 * z + _S9) * z + _S7) * z + _S5) * z + _S3) * z + _S1
    ps = ps * r
    pc = ((((_C10 * z + _C8) * z + _C6) * z + _C4) * z + _C2) * z + _C0
    an = jnp.abs(n)
    sgn = 1.0 - 2.0 * (an - 2.0 * jnp.floor(an * 0.5))
    return ps * sgn, pc * sgn


def _volume_body(xyzT_ref, ynm_ref, scale_ref, off_ref, w1_ref, b1_ref,
                 w2_ref, b2_ref, dT_ref, cT_ref):
    B = _B
    C = B // _G
    t = xyzT_ref[:] * scale_ref[:] + off_ref[:]
    inb = (t >= -1.0) & (t <= 1.0)
    mask = inb[0:1] & inb[1:2] & inb[2:3]

    s, c = _sincos_pi(t)
    feats = [t, s, c]
    for _ in range(1, _PE_L):
        s, c = 2.0 * s * c, 1.0 - 2.0 * s * s
        feats.append(s)
        feats.append(c)
    pe = jnp.concatenate(feats, axis=0)

    pe_g = jnp.concatenate([pe[:, i * C:(i + 1) * C] for i in range(_G)],
                           axis=0).astype(jnp.bfloat16)
    f_g = jnp.dot(w1_ref[:], pe_g,
                  preferred_element_type=jnp.float32)
    f_g = jnp.maximum(f_g + b1_ref[:], 0.0)

    ynmT = ynm_ref[:].T.astype(jnp.bfloat16)
    z_parts = []
    for i in range(_G):
        z_parts.append(f_g[_H * i:_H * (i + 1), :].astype(jnp.bfloat16))
        z_parts.append(ynmT[:, i * C:(i + 1) * C])
    z_g = jnp.concatenate(z_parts, axis=0)
    o_g = jnp.dot(w2_ref[:], z_g,
                  preferred_element_type=jnp.float32) + b2_ref[:]
    o = jnp.concatenate([o_g[4 * i:4 * (i + 1), :] for i in range(_G)],
                        axis=1)

    od = o[0:1, :]
    sp = jnp.maximum(od, 0.0) + jnp.log1p(jnp.exp(-jnp.abs(od)))
    oc = o[1:4, :]
    sig = 1.0 / (1.0 + jnp.exp(-oc))
    dT_ref[:] = jnp.where(mask, sp, 0.0)
    cT_ref[:] = jnp.where(mask, sig, 0.0)


def kernel(xyz, ynm, aabb, W1, b1, Wd, bd, Wc, bc):
    N, S, _ = xyz.shape
    P = N * S
    B, G = _B, _G

    xyzT = xyz.transpose(2, 0, 1).reshape(3, P)
    ynm2 = ynm.reshape(P, 16)
    rng = aabb[1] - aabb[0]
    scale = (2.0 / rng).reshape(3, 1)
    off = (-2.0 * aabb[0] / rng - 1.0).reshape(3, 1)

    W1T = W1.T
    w1bd = jax.scipy.linalg.block_diag(*([W1T] * G)).astype(jnp.bfloat16)
    b1t = jnp.tile(b1, G).reshape(G * _H, 1)
    W2 = jnp.concatenate(
        [jnp.concatenate([Wd, jnp.zeros((16, 1), jnp.float32)], axis=0), Wc],
        axis=1)
    w2bd = jax.scipy.linalg.block_diag(*([W2.T] * G)).astype(jnp.bfloat16)
    b2t = jnp.tile(jnp.concatenate([bd, bc]), G).reshape(4 * G, 1)

    grid = P // B
    dT, cT = pl.pallas_call(
        _volume_body,
        grid=(grid,),
        in_specs=[
            pl.BlockSpec((3, B), lambda i: (0, i)),
            pl.BlockSpec((B, 16), lambda i: (i, 0)),
            pl.BlockSpec((3, 1), lambda i: (0, 0)),
            pl.BlockSpec((3, 1), lambda i: (0, 0)),
            pl.BlockSpec(w1bd.shape, lambda i: (0, 0)),
            pl.BlockSpec((G * _H, 1), lambda i: (0, 0)),
            pl.BlockSpec(w2bd.shape, lambda i: (0, 0)),
            pl.BlockSpec((4 * G, 1), lambda i: (0, 0)),
        ],
        out_specs=[
            pl.BlockSpec((1, B), lambda i: (0, i)),
            pl.BlockSpec((3, B), lambda i: (0, i)),
        ],
        out_shape=[
            jax.ShapeDtypeStruct((1, P), jnp.float32),
            jax.ShapeDtypeStruct((3, P), jnp.float32),
        ],
    )(xyzT, ynm2, scale, off, w1bd, b1t, w2bd, b2t)

    density = dT.reshape(N, S, 1)
    color = cT.reshape(3, N, S).transpose(1, 2, 0)
    return density, color
